# baseline (speedup 1.0000x reference)
import functools

import jax
import jax.numpy as jnp
from jax import lax
from jax.experimental import pallas as pl
from jax.experimental.pallas import tpu as pltpu
from jax.experimental.pallas import tpu_sc as plsc

_INFO = plsc.get_sparse_core_info()
_NC, _NS = _INFO.num_cores, _INFO.num_subcores
_NW = _NC * _NS

_N, _M = 16384, 200
_D = 64
_ROWS_W = _N // _NW
_RC = 4
_N_CHUNKS = _ROWS_W // _RC


def _embed_kernel(x_hbm, table_hbm, out_hbm, table_v, idx_v, rows_v,
                  outsem0, outsem1):
    wid = lax.axis_index("s") * _NC + lax.axis_index("c")
    base = wid * _ROWS_W
    pltpu.sync_copy(table_hbm, table_v)

    def assemble(rows_ref):
        def body(k, carry):
            b = jnp.minimum(k * 16, _M - 16)
            for sub in range(_RC):
                vec = idx_v[sub, pl.ds(b, 16)]
                for half in range(2):
                    loads = []
                    for u in range(8):
                        s = vec[half * 8 + u]
                        loads.append(
                            [table_v[s, pl.ds(16 * g, 16)]
                             for g in range(_D // 16)])
                    for u in range(8):
                        r = b + half * 8 + u
                        for g in range(_D // 16):
                            rows_ref[sub, r, pl.ds(16 * g, 16)] = loads[u][g]
            return carry
        lax.fori_loop(0, (_M + 15) // 16, body, 0)

    def chunk_step(j, buf, sem):
        g = j * 2 + buf
        off = base + g * _RC
        rows_ref = rows_v.at[buf]
        pltpu.sync_copy(x_hbm.at[pl.ds(off, _RC)], idx_v)

        @pl.when(j >= 1)
        def _():
            pltpu.make_async_copy(
                rows_ref, out_hbm.at[pl.ds(off, _RC)], sem).wait()

        assemble(rows_ref)
        pltpu.async_copy(rows_ref, out_hbm.at[pl.ds(off, _RC)], sem)

    def outer(j, carry):
        chunk_step(j, 0, outsem0)
        chunk_step(j, 1, outsem1)
        return carry

    lax.fori_loop(0, _N_CHUNKS // 2, outer, 0)

    pltpu.make_async_copy(
        rows_v.at[0], out_hbm.at[pl.ds(base, _RC)], outsem0).wait()
    pltpu.make_async_copy(
        rows_v.at[1], out_hbm.at[pl.ds(base, _RC)], outsem1).wait()


@jax.jit
def kernel(x, table):
    mesh = plsc.VectorSubcoreMesh(core_axis_name="c", subcore_axis_name="s")
    run = functools.partial(
        pl.kernel,
        mesh=mesh,
        out_type=jax.ShapeDtypeStruct((_N, _M, _D), jnp.float32),
        scratch_types=[
            pltpu.VMEM((8, _D), jnp.float32),
            pltpu.VMEM((_RC, _M), jnp.int32),
            pltpu.VMEM((2, _RC, _M, _D), jnp.float32),
            pltpu.SemaphoreType.DMA,
            pltpu.SemaphoreType.DMA,
        ],
        compiler_params=pltpu.CompilerParams(use_tc_tiling_on_sc=False),
    )(_embed_kernel)
    return run(x, table)
